# Initial kernel scaffold; baseline (speedup 1.0000x reference)
import jax
import jax.numpy as jnp
from jax.experimental import pallas as pl


def kernel(x, edge_index, edge_attr, batch, W_atom, b_atom, W_bond, b_bond, eps, Wm1, bm1, Wm2, bm2, W_att, b_att, W_h1, b_h1, W_h2, b_h2):
    raise NotImplementedError("write your pallas kernel here")


# speedup vs baseline: 2.9854x; 2.9854x over previous
import functools

import jax
import jax.numpy as jnp
from jax import lax
from jax.experimental import pallas as pl
from jax.experimental.pallas import tpu as pltpu
from jax.experimental.pallas import tpu_sc as plsc

_EC = 128
_NSUB = 16
_NCORE = 2


def _msg_body(h_hbm, e_hbm, src_hbm, dst_hbm, agg_hbm,
              acc_sh, src_v, dst_v, gath_v, e_v, zero_v, sem):
    cid = lax.axis_index("c")
    sid = lax.axis_index("s")
    n = h_hbm.shape[0]
    zr = zero_v.shape[0]
    ngroups = n // zr
    giters = (ngroups + _NSUB - 1) // _NSUB

    def zrow(i, c):
        for j in range(8):
            s = pl.ds(j * 16, 16)
            zero_v[i, s] = jnp.zeros((16,), jnp.float32)
        return c
    lax.fori_loop(0, zr, zrow, 0)

    def zcopy(u, c):
        g = u * _NSUB + sid

        @pl.when(g < ngroups)
        def _():
            pltpu.sync_copy(zero_v, acc_sh.at[pl.ds(g * zr, zr)])
        return c
    lax.fori_loop(0, giters, zcopy, 0)
    plsc.subcore_barrier()

    nchunks = src_hbm.shape[0] // _EC
    per_core = nchunks // _NCORE
    iters = (per_core + _NSUB - 1) // _NSUB

    def chunk_body(u, c):
        t = u * _NSUB + sid

        @pl.when(t < per_core)
        def _():
            eoff = (cid * per_core + t) * _EC
            pltpu.sync_copy(src_hbm.at[pl.ds(eoff, _EC)], src_v)
            pltpu.sync_copy(dst_hbm.at[pl.ds(eoff, _EC)], dst_v)
            pltpu.async_copy(h_hbm.at[src_v], gath_v, sem).wait()
            pltpu.sync_copy(e_hbm.at[pl.ds(eoff, _EC)], e_v)

            def crow(i, cc):
                for j in range(8):
                    s = pl.ds(j * 16, 16)
                    gath_v[i, s] = jnp.maximum(gath_v[i, s] + e_v[i, s], 0.0)
                return cc
            lax.fori_loop(0, _EC, crow, 0)
            pltpu.sync_copy(gath_v, acc_sh.at[dst_v], add=True)
        return c
    lax.fori_loop(0, iters, chunk_body, 0)
    plsc.subcore_barrier()

    def wcopy(u, c):
        g = u * _NSUB + sid

        @pl.when(g < ngroups)
        def _():
            pltpu.sync_copy(acc_sh.at[pl.ds(g * zr, zr)],
                            agg_hbm.at[cid, pl.ds(g * zr, zr)])
        return c
    lax.fori_loop(0, giters, wcopy, 0)


def _msg_call(h, e, src, dst):
    n, hd = h.shape
    mesh = plsc.VectorSubcoreMesh(core_axis_name="c", subcore_axis_name="s")
    f = pl.kernel(
        _msg_body,
        out_type=jax.ShapeDtypeStruct((_NCORE, n, hd), jnp.float32),
        mesh=mesh,
        scratch_types=[
            pltpu.VMEM_SHARED((n, hd), jnp.float32),
            pltpu.VMEM((_EC,), jnp.int32),
            pltpu.VMEM((_EC,), jnp.int32),
            pltpu.VMEM((_EC, hd), jnp.float32),
            pltpu.VMEM((_EC, hd), jnp.float32),
            pltpu.VMEM((80, hd), jnp.float32),
            pltpu.SemaphoreType.DMA,
        ],
    )
    return f(h, e, src, dst)


def _proj_body(x_ref, w_ref, b_ref, o_ref):
    o_ref[...] = jnp.maximum(
        jnp.dot(x_ref[...], w_ref[...], preferred_element_type=jnp.float32)
        + b_ref[...], 0.0)


def _proj(x, w, b, blk):
    m, k = x.shape
    kd, hd = w.shape
    grid = m // blk
    return pl.pallas_call(
        _proj_body,
        grid=(grid,),
        in_specs=[
            pl.BlockSpec((blk, k), lambda i: (i, 0)),
            pl.BlockSpec((k, hd), lambda i: (0, 0)),
            pl.BlockSpec((1, hd), lambda i: (0, 0)),
        ],
        out_specs=pl.BlockSpec((blk, hd), lambda i: (i, 0)),
        out_shape=jax.ShapeDtypeStruct((m, hd), jnp.float32),
    )(x, w, b)


def _mlp_body(eps_ref, h_ref, a_ref, w1_ref, b1_ref, w2_ref, b2_ref, o_ref):
    hh = (1.0 + eps_ref[0]) * h_ref[...] + a_ref[0] + a_ref[1]
    t = jnp.maximum(
        jnp.dot(hh, w1_ref[...], preferred_element_type=jnp.float32)
        + b1_ref[...], 0.0)
    t = jnp.dot(t, w2_ref[...], preferred_element_type=jnp.float32) + b2_ref[...]
    o_ref[...] = jnp.maximum(t, 0.0)


def _mlp(eps_l, h, agg, w1, b1, w2, b2, blk):
    n, hd = h.shape
    return pl.pallas_call(
        _mlp_body,
        grid=(n // blk,),
        in_specs=[
            pl.BlockSpec(memory_space=pltpu.SMEM),
            pl.BlockSpec((blk, hd), lambda i: (i, 0)),
            pl.BlockSpec((_NCORE, blk, hd), lambda i: (0, i, 0)),
            pl.BlockSpec((hd, hd), lambda i: (0, 0)),
            pl.BlockSpec((1, hd), lambda i: (0, 0)),
            pl.BlockSpec((hd, hd), lambda i: (0, 0)),
            pl.BlockSpec((1, hd), lambda i: (0, 0)),
        ],
        out_specs=pl.BlockSpec((blk, hd), lambda i: (i, 0)),
        out_shape=jax.ShapeDtypeStruct((n, hd), jnp.float32),
    )(eps_l, h, agg, w1, b1, w2, b2)


_NEG = -1e30


def _r1_body(h_ref, batch_ref, watt_ref, batt_ref, logits_ref, lmax_ref):
    i = pl.program_id(0)
    nb = lmax_ref.shape[1]
    lg = (jnp.dot(h_ref[...], watt_ref[...], preferred_element_type=jnp.float32)
          + batt_ref[...])
    logits_ref[...] = lg
    biota = lax.broadcasted_iota(jnp.int32, (1, nb), 1)
    mask = batch_ref[...] == biota
    mm = jnp.where(mask, lg, _NEG)
    blkmax = jnp.max(mm, axis=0, keepdims=True)

    @pl.when(i == 0)
    def _():
        lmax_ref[...] = jnp.full(lmax_ref.shape, _NEG, jnp.float32)
    lmax_ref[...] = jnp.maximum(lmax_ref[...], blkmax)


def _r2_body(logits_ref, batch_ref, lmax_ref, ex_ref, denom_ref):
    i = pl.program_id(0)
    nb = lmax_ref.shape[1]
    biota = lax.broadcasted_iota(jnp.int32, (1, nb), 1)
    mask = batch_ref[...] == biota
    lmax_pn = jnp.max(jnp.where(mask, lmax_ref[...], _NEG), axis=1,
                      keepdims=True)
    ex = jnp.exp(logits_ref[...] - lmax_pn)
    ex_ref[...] = ex
    part = jnp.sum(jnp.where(mask, ex, 0.0), axis=0, keepdims=True)

    @pl.when(i == 0)
    def _():
        denom_ref[...] = jnp.zeros(denom_ref.shape, jnp.float32)
    denom_ref[...] += part


def _r3_body(h_ref, ex_ref, batch_ref, denom_ref, ge_ref):
    i = pl.program_id(0)
    nb = denom_ref.shape[1]
    biota = lax.broadcasted_iota(jnp.int32, (1, nb), 1)
    mask = batch_ref[...] == biota
    denom_pn = jnp.sum(jnp.where(mask, denom_ref[...], 0.0), axis=1,
                       keepdims=True)
    alpha = ex_ref[...] / (denom_pn + 1e-16)
    hw = h_ref[...] * alpha
    mask_f = jnp.where(mask, 1.0, 0.0)
    part = lax.dot_general(mask_f, hw, (((0,), (0,)), ((), ())),
                           preferred_element_type=jnp.float32)

    @pl.when(i == 0)
    def _():
        ge_ref[...] = jnp.zeros(ge_ref.shape, jnp.float32)
    ge_ref[...] += part


def _head_body(ge_ref, w1_ref, b1_ref, w2_ref, b2_ref, pred_ref):
    z = jnp.maximum(
        jnp.dot(ge_ref[...], w1_ref[...], preferred_element_type=jnp.float32)
        + b1_ref[...], 0.0)
    pred_ref[...] = (jnp.dot(z, w2_ref[...], preferred_element_type=jnp.float32)
                     + b2_ref[...])


def _readout(h, batch2, w_att, b_att, w_h1, b_h1, w_h2, b_h2, blk):
    n, hd = h.shape
    nb = 256
    grid = n // blk
    logits, lmax = pl.pallas_call(
        _r1_body,
        grid=(grid,),
        in_specs=[
            pl.BlockSpec((blk, hd), lambda i: (i, 0)),
            pl.BlockSpec((blk, 1), lambda i: (i, 0)),
            pl.BlockSpec((hd, 1), lambda i: (0, 0)),
            pl.BlockSpec((1, 1), lambda i: (0, 0)),
        ],
        out_specs=[
            pl.BlockSpec((blk, 1), lambda i: (i, 0)),
            pl.BlockSpec((1, nb), lambda i: (0, 0)),
        ],
        out_shape=[
            jax.ShapeDtypeStruct((n, 1), jnp.float32),
            jax.ShapeDtypeStruct((1, nb), jnp.float32),
        ],
    )(h, batch2, w_att, b_att)

    ex, denom = pl.pallas_call(
        _r2_body,
        grid=(grid,),
        in_specs=[
            pl.BlockSpec((blk, 1), lambda i: (i, 0)),
            pl.BlockSpec((blk, 1), lambda i: (i, 0)),
            pl.BlockSpec((1, nb), lambda i: (0, 0)),
        ],
        out_specs=[
            pl.BlockSpec((blk, 1), lambda i: (i, 0)),
            pl.BlockSpec((1, nb), lambda i: (0, 0)),
        ],
        out_shape=[
            jax.ShapeDtypeStruct((n, 1), jnp.float32),
            jax.ShapeDtypeStruct((1, nb), jnp.float32),
        ],
    )(logits, batch2, lmax)

    ge = pl.pallas_call(
        _r3_body,
        grid=(grid,),
        in_specs=[
            pl.BlockSpec((blk, hd), lambda i: (i, 0)),
            pl.BlockSpec((blk, 1), lambda i: (i, 0)),
            pl.BlockSpec((blk, 1), lambda i: (i, 0)),
            pl.BlockSpec((1, nb), lambda i: (0, 0)),
        ],
        out_specs=pl.BlockSpec((nb, hd), lambda i: (0, 0)),
        out_shape=jax.ShapeDtypeStruct((nb, hd), jnp.float32),
    )(h, ex, batch2, denom)

    hh = w_h1.shape[1]
    pred = pl.pallas_call(
        _head_body,
        out_shape=jax.ShapeDtypeStruct((nb, w_h2.shape[1]), jnp.float32),
    )(ge, w_h1, b_h1.reshape(1, hh), w_h2, b_h2.reshape(1, w_h2.shape[1]))
    return pred, ge


def kernel(x, edge_index, edge_attr, batch, W_atom, b_atom, W_bond, b_bond,
           eps, Wm1, bm1, Wm2, bm2, W_att, b_att, W_h1, b_h1, W_h2, b_h2):
    n, af = x.shape
    e_num, bf = edge_attr.shape
    hd = W_atom.shape[1]
    L = Wm1.shape[0]

    src = edge_index[0].astype(jnp.int32)
    dst = edge_index[1].astype(jnp.int32)

    afp = (af + 15) // 16 * 16
    bfp = (bf + 15) // 16 * 16
    xp = jnp.pad(x, ((0, 0), (0, afp - af)))
    wap = jnp.pad(W_atom, ((0, afp - af), (0, 0)))
    eap = jnp.pad(edge_attr, ((0, 0), (0, bfp - bf)))
    wbp = jnp.pad(W_bond, ((0, bfp - bf), (0, 0)))

    h = _proj(xp, wap, b_atom.reshape(1, hd), blk=1000)
    e = _proj(eap, wbp, b_bond.reshape(1, hd), blk=2000)

    for l in range(L):
        agg = _msg_call(h, e, src, dst)
        h = _mlp(eps[l].reshape(1), h, agg, Wm1[l], bm1[l].reshape(1, hd),
                 Wm2[l], bm2[l].reshape(1, hd), blk=1000)

    node_emb = h
    batch2 = batch.astype(jnp.int32).reshape(n, 1)
    pred, graph_emb = _readout(h, batch2, W_att, b_att.reshape(1, 1),
                               W_h1, b_h1, W_h2, b_h2, blk=1000)
    return pred, node_emb, graph_emb


# speedup vs baseline: 3.8702x; 1.2964x over previous
import functools

import jax
import jax.numpy as jnp
from jax import lax
from jax.experimental import pallas as pl
from jax.experimental.pallas import tpu as pltpu
from jax.experimental.pallas import tpu_sc as plsc

_EC = 128
_NSUB = 16
_NCORE = 2


def _msg_body(h_hbm, e_hbm, src_hbm, dst_hbm, agg_hbm,
              acc_sh, src_v, dst_v, gath_v, e_v, zero_v, sem, sem_e,
              sem_i0, sem_i1):
    cid = lax.axis_index("c")
    sid = lax.axis_index("s")
    n = h_hbm.shape[0]
    zr = zero_v.shape[0]
    ngroups = n // zr
    giters = (ngroups + _NSUB - 1) // _NSUB

    def zrow(i, c):
        for j in range(8):
            s = pl.ds(j * 16, 16)
            zero_v[i, s] = jnp.zeros((16,), jnp.float32)
        return c
    lax.fori_loop(0, zr, zrow, 0)

    def zcopy(u, c):
        g = u * _NSUB + sid

        @pl.when(g < ngroups)
        def _():
            pltpu.sync_copy(zero_v, acc_sh.at[pl.ds(g * zr, zr)])
        return c
    lax.fori_loop(0, giters, zcopy, 0)
    plsc.subcore_barrier()

    nchunks = src_hbm.shape[0] // _EC
    per_core = nchunks // _NCORE
    iters = (per_core + _NSUB - 1) // _NSUB

    def chunk_body(u, c):
        t = u * _NSUB + sid

        @pl.when(t < per_core)
        def _():
            eoff = (cid * per_core + t) * _EC
            ce = pltpu.async_copy(e_hbm.at[pl.ds(eoff, _EC)], e_v, sem_e)
            cs = pltpu.async_copy(src_hbm.at[pl.ds(eoff, _EC)], src_v, sem_i0)
            cd = pltpu.async_copy(dst_hbm.at[pl.ds(eoff, _EC)], dst_v, sem_i1)
            cs.wait()
            cg = pltpu.async_copy(h_hbm.at[src_v], gath_v, sem)
            ce.wait()
            cg.wait()
            cd.wait()

            def crow(i, cc):
                for j in range(8):
                    s = pl.ds(j * 16, 16)
                    gath_v[i, s] = jnp.maximum(gath_v[i, s] + e_v[i, s], 0.0)
                return cc
            lax.fori_loop(0, _EC, crow, 0)
            pltpu.sync_copy(gath_v, acc_sh.at[dst_v], add=True)
        return c
    lax.fori_loop(0, iters, chunk_body, 0)
    plsc.subcore_barrier()

    def wcopy(u, c):
        g = u * _NSUB + sid

        @pl.when(g < ngroups)
        def _():
            pltpu.sync_copy(acc_sh.at[pl.ds(g * zr, zr)],
                            agg_hbm.at[cid, pl.ds(g * zr, zr)])
        return c
    lax.fori_loop(0, giters, wcopy, 0)


def _msg_call(h, e, src, dst):
    n, hd = h.shape
    mesh = plsc.VectorSubcoreMesh(core_axis_name="c", subcore_axis_name="s")
    f = pl.kernel(
        _msg_body,
        out_type=jax.ShapeDtypeStruct((_NCORE, n, hd), jnp.float32),
        mesh=mesh,
        scratch_types=[
            pltpu.VMEM_SHARED((n, hd), jnp.float32),
            pltpu.VMEM((_EC,), jnp.int32),
            pltpu.VMEM((_EC,), jnp.int32),
            pltpu.VMEM((_EC, hd), jnp.float32),
            pltpu.VMEM((_EC, hd), jnp.float32),
            pltpu.VMEM((80, hd), jnp.float32),
            pltpu.SemaphoreType.DMA,
            pltpu.SemaphoreType.DMA,
            pltpu.SemaphoreType.DMA,
            pltpu.SemaphoreType.DMA,
        ],
    )
    return f(h, e, src, dst)


def _proj_body(x_ref, w_ref, b_ref, o_ref):
    o_ref[...] = jnp.maximum(
        jnp.dot(x_ref[...], w_ref[...], preferred_element_type=jnp.float32)
        + b_ref[...], 0.0)


def _proj(x, w, b, blk):
    m, k = x.shape
    kd, hd = w.shape
    grid = m // blk
    return pl.pallas_call(
        _proj_body,
        grid=(grid,),
        in_specs=[
            pl.BlockSpec((blk, k), lambda i: (i, 0)),
            pl.BlockSpec((k, hd), lambda i: (0, 0)),
            pl.BlockSpec((1, hd), lambda i: (0, 0)),
        ],
        out_specs=pl.BlockSpec((blk, hd), lambda i: (i, 0)),
        out_shape=jax.ShapeDtypeStruct((m, hd), jnp.float32),
    )(x, w, b)


def _mlp_body(eps_ref, h_ref, a_ref, w1_ref, b1_ref, w2_ref, b2_ref, o_ref):
    hh = (1.0 + eps_ref[0]) * h_ref[...] + a_ref[0] + a_ref[1]
    t = jnp.maximum(
        jnp.dot(hh, w1_ref[...], preferred_element_type=jnp.float32)
        + b1_ref[...], 0.0)
    t = jnp.dot(t, w2_ref[...], preferred_element_type=jnp.float32) + b2_ref[...]
    o_ref[...] = jnp.maximum(t, 0.0)


def _mlp(eps_l, h, agg, w1, b1, w2, b2, blk):
    n, hd = h.shape
    return pl.pallas_call(
        _mlp_body,
        grid=(n // blk,),
        in_specs=[
            pl.BlockSpec(memory_space=pltpu.SMEM),
            pl.BlockSpec((blk, hd), lambda i: (i, 0)),
            pl.BlockSpec((_NCORE, blk, hd), lambda i: (0, i, 0)),
            pl.BlockSpec((hd, hd), lambda i: (0, 0)),
            pl.BlockSpec((1, hd), lambda i: (0, 0)),
            pl.BlockSpec((hd, hd), lambda i: (0, 0)),
            pl.BlockSpec((1, hd), lambda i: (0, 0)),
        ],
        out_specs=pl.BlockSpec((blk, hd), lambda i: (i, 0)),
        out_shape=jax.ShapeDtypeStruct((n, hd), jnp.float32),
    )(eps_l, h, agg, w1, b1, w2, b2)


_NEG = -1e30


def _r1_body(h_ref, batch_ref, watt_ref, batt_ref, logits_ref, lmax_ref):
    i = pl.program_id(0)
    nb = lmax_ref.shape[1]
    lg = (jnp.dot(h_ref[...], watt_ref[...], preferred_element_type=jnp.float32)
          + batt_ref[...])
    logits_ref[...] = lg
    biota = lax.broadcasted_iota(jnp.int32, (1, nb), 1)
    mask = batch_ref[...] == biota
    mm = jnp.where(mask, lg, _NEG)
    blkmax = jnp.max(mm, axis=0, keepdims=True)

    @pl.when(i == 0)
    def _():
        lmax_ref[...] = jnp.full(lmax_ref.shape, _NEG, jnp.float32)
    lmax_ref[...] = jnp.maximum(lmax_ref[...], blkmax)


def _r2_body(logits_ref, batch_ref, lmax_ref, ex_ref, denom_ref):
    i = pl.program_id(0)
    nb = lmax_ref.shape[1]
    biota = lax.broadcasted_iota(jnp.int32, (1, nb), 1)
    mask = batch_ref[...] == biota
    lmax_pn = jnp.max(jnp.where(mask, lmax_ref[...], _NEG), axis=1,
                      keepdims=True)
    ex = jnp.exp(logits_ref[...] - lmax_pn)
    ex_ref[...] = ex
    part = jnp.sum(jnp.where(mask, ex, 0.0), axis=0, keepdims=True)

    @pl.when(i == 0)
    def _():
        denom_ref[...] = jnp.zeros(denom_ref.shape, jnp.float32)
    denom_ref[...] += part


def _r3_body(h_ref, ex_ref, batch_ref, denom_ref, ge_ref):
    i = pl.program_id(0)
    nb = denom_ref.shape[1]
    biota = lax.broadcasted_iota(jnp.int32, (1, nb), 1)
    mask = batch_ref[...] == biota
    denom_pn = jnp.sum(jnp.where(mask, denom_ref[...], 0.0), axis=1,
                       keepdims=True)
    alpha = ex_ref[...] / (denom_pn + 1e-16)
    hw = h_ref[...] * alpha
    mask_f = jnp.where(mask, 1.0, 0.0)
    part = lax.dot_general(mask_f, hw, (((0,), (0,)), ((), ())),
                           preferred_element_type=jnp.float32)

    @pl.when(i == 0)
    def _():
        ge_ref[...] = jnp.zeros(ge_ref.shape, jnp.float32)
    ge_ref[...] += part


def _head_body(ge_ref, w1_ref, b1_ref, w2_ref, b2_ref, pred_ref):
    z = jnp.maximum(
        jnp.dot(ge_ref[...], w1_ref[...], preferred_element_type=jnp.float32)
        + b1_ref[...], 0.0)
    pred_ref[...] = (jnp.dot(z, w2_ref[...], preferred_element_type=jnp.float32)
                     + b2_ref[...])


def _readout(h, batch2, w_att, b_att, w_h1, b_h1, w_h2, b_h2, blk):
    n, hd = h.shape
    nb = 256
    grid = n // blk
    logits, lmax = pl.pallas_call(
        _r1_body,
        grid=(grid,),
        in_specs=[
            pl.BlockSpec((blk, hd), lambda i: (i, 0)),
            pl.BlockSpec((blk, 1), lambda i: (i, 0)),
            pl.BlockSpec((hd, 1), lambda i: (0, 0)),
            pl.BlockSpec((1, 1), lambda i: (0, 0)),
        ],
        out_specs=[
            pl.BlockSpec((blk, 1), lambda i: (i, 0)),
            pl.BlockSpec((1, nb), lambda i: (0, 0)),
        ],
        out_shape=[
            jax.ShapeDtypeStruct((n, 1), jnp.float32),
            jax.ShapeDtypeStruct((1, nb), jnp.float32),
        ],
    )(h, batch2, w_att, b_att)

    ex, denom = pl.pallas_call(
        _r2_body,
        grid=(grid,),
        in_specs=[
            pl.BlockSpec((blk, 1), lambda i: (i, 0)),
            pl.BlockSpec((blk, 1), lambda i: (i, 0)),
            pl.BlockSpec((1, nb), lambda i: (0, 0)),
        ],
        out_specs=[
            pl.BlockSpec((blk, 1), lambda i: (i, 0)),
            pl.BlockSpec((1, nb), lambda i: (0, 0)),
        ],
        out_shape=[
            jax.ShapeDtypeStruct((n, 1), jnp.float32),
            jax.ShapeDtypeStruct((1, nb), jnp.float32),
        ],
    )(logits, batch2, lmax)

    ge = pl.pallas_call(
        _r3_body,
        grid=(grid,),
        in_specs=[
            pl.BlockSpec((blk, hd), lambda i: (i, 0)),
            pl.BlockSpec((blk, 1), lambda i: (i, 0)),
            pl.BlockSpec((blk, 1), lambda i: (i, 0)),
            pl.BlockSpec((1, nb), lambda i: (0, 0)),
        ],
        out_specs=pl.BlockSpec((nb, hd), lambda i: (0, 0)),
        out_shape=jax.ShapeDtypeStruct((nb, hd), jnp.float32),
    )(h, ex, batch2, denom)

    hh = w_h1.shape[1]
    pred = pl.pallas_call(
        _head_body,
        out_shape=jax.ShapeDtypeStruct((nb, w_h2.shape[1]), jnp.float32),
    )(ge, w_h1, b_h1.reshape(1, hh), w_h2, b_h2.reshape(1, w_h2.shape[1]))
    return pred, ge


def kernel(x, edge_index, edge_attr, batch, W_atom, b_atom, W_bond, b_bond,
           eps, Wm1, bm1, Wm2, bm2, W_att, b_att, W_h1, b_h1, W_h2, b_h2):
    n, af = x.shape
    e_num, bf = edge_attr.shape
    hd = W_atom.shape[1]
    L = Wm1.shape[0]

    src = edge_index[0].astype(jnp.int32)
    dst = edge_index[1].astype(jnp.int32)

    afp = (af + 15) // 16 * 16
    bfp = (bf + 15) // 16 * 16
    xp = jnp.pad(x, ((0, 0), (0, afp - af)))
    wap = jnp.pad(W_atom, ((0, afp - af), (0, 0)))
    eap = jnp.pad(edge_attr, ((0, 0), (0, bfp - bf)))
    wbp = jnp.pad(W_bond, ((0, bfp - bf), (0, 0)))

    h = _proj(xp, wap, b_atom.reshape(1, hd), blk=1000)
    e = _proj(eap, wbp, b_bond.reshape(1, hd), blk=2000)

    for l in range(L):
        agg = _msg_call(h, e, src, dst)
        h = _mlp(eps[l].reshape(1), h, agg, Wm1[l], bm1[l].reshape(1, hd),
                 Wm2[l], bm2[l].reshape(1, hd), blk=1000)

    node_emb = h
    batch2 = batch.astype(jnp.int32).reshape(n, 1)
    pred, graph_emb = _readout(h, batch2, W_att, b_att.reshape(1, 1),
                               W_h1, b_h1, W_h2, b_h2, blk=1000)
    return pred, node_emb, graph_emb
